# Initial kernel scaffold; baseline (speedup 1.0000x reference)
import jax
import jax.numpy as jnp
from jax.experimental import pallas as pl


def kernel(z, prior_samples):
    raise NotImplementedError("write your pallas kernel here")


# speedup vs baseline: 4.9280x; 4.9280x over previous
import math

import jax
import jax.numpy as jnp
from jax import lax
from jax.experimental import pallas as pl
from jax.experimental.pallas import tpu as pltpu

_DIM = 8
_CB = 512
_LOGVAR_MIN = -30.0
_LOGVAR_MAX = 20.0
_KL_SCALE = 1.4426 * 0.5


def _body(z_ref, w_ref, g_ref, zhat_ref, idx_ref, kl_ref):
    zb = z_ref[0]
    mu = zb[0:16, :]
    lv = zb[16:32, :]
    lvc = jnp.clip(lv, _LOGVAR_MIN, _LOGVAR_MAX)
    var = jnp.exp(lvc)
    inv = 1.0 / var
    a = 0.5 - 0.5 * inv
    b = mu * inv
    acts = jnp.concatenate([a, b], axis=0)
    score = lax.dot_general(
        w_ref[...], acts, (((1,), (0,)), ((), ())),
        preferred_element_type=jnp.float32,
        precision=lax.Precision.HIGHEST)
    blk = score.shape[1]
    iota = lax.broadcasted_iota(jnp.int32, (_CB, blk), 0)
    s0 = score[0:_CB, :]
    s1 = score[_CB:2 * _CB, :]
    m0 = jnp.max(s0, axis=0, keepdims=True)
    m1 = jnp.max(s1, axis=0, keepdims=True)
    idx0 = jnp.min(jnp.where(s0 == m0, iota, _CB), axis=0)
    idx1 = jnp.min(jnp.where(s1 == m1, iota, _CB), axis=0)
    idx_ref[0, 0, :] = idx0
    idx_ref[0, 1, :] = idx1
    onehot = jnp.concatenate(
        [(iota == idx0[None, :]).astype(jnp.float32),
         (iota == idx1[None, :]).astype(jnp.float32)], axis=0)
    zhat_ref[0] = lax.dot_general(
        g_ref[...], onehot, (((1,), (0,)), ((), ())),
        preferred_element_type=jnp.float32,
        precision=lax.Precision.HIGHEST)
    part = jnp.sum(mu * mu + var - 1.0 - lvc)

    @pl.when(jnp.logical_and(pl.program_id(0) == 0, pl.program_id(1) == 0))
    def _init():
        kl_ref[...] = jnp.zeros_like(kl_ref)

    kl_ref[...] += part


def _codebook_mats(prior_samples):
    ps = prior_samples.astype(jnp.float32)
    ps2 = ps * ps
    j = jnp.arange(16)
    dsel = j // 2
    par = j % 2
    gsel = jnp.arange(2)[:, None, None]
    wa = jnp.where(par[None, None, :] == gsel, ps2[:, dsel][None], 0.0)
    wb = jnp.where(par[None, None, :] == gsel, ps[:, dsel][None], 0.0)
    wa = wa.reshape(2 * _CB, 16)
    wb = wb.reshape(2 * _CB, 16)
    w_score = jnp.concatenate([wa, wb], axis=1)
    g_gather = wb.T
    return w_score, g_gather


def kernel(z, prior_samples):
    batch, chans, hh, ww = z.shape
    spatial = hh * ww
    blk = 512
    zr = z.reshape(batch, chans, spatial)
    w_score, g_gather = _codebook_mats(prior_samples)
    grid = (batch, spatial // blk)
    zhat3, idx3, klsum = pl.pallas_call(
        _body,
        grid=grid,
        in_specs=[
            pl.BlockSpec((1, chans, blk), lambda b, s: (b, 0, s)),
            pl.BlockSpec((2 * _CB, 2 * 16), lambda b, s: (0, 0)),
            pl.BlockSpec((16, 2 * _CB), lambda b, s: (0, 0)),
        ],
        out_specs=[
            pl.BlockSpec((1, 16, blk), lambda b, s: (b, 0, s)),
            pl.BlockSpec((1, 2, blk), lambda b, s: (b, 0, s)),
            pl.BlockSpec((1, 1), lambda b, s: (0, 0)),
        ],
        out_shape=[
            jax.ShapeDtypeStruct((batch, 16, spatial), jnp.float32),
            jax.ShapeDtypeStruct((batch, 2, spatial), jnp.int32),
            jax.ShapeDtypeStruct((1, 1), jnp.float32),
        ],
    )(zr, w_score, g_gather)
    zhat = zhat3.reshape(batch, 16, hh, ww)
    indices = idx3.reshape(batch, 2, hh, ww)
    kl_loss = klsum[0, 0] * jnp.float32(_KL_SCALE / (batch * spatial * 2))
    return (zhat, kl_loss, indices)


# speedup vs baseline: 8.0398x; 1.6314x over previous
import jax
import jax.numpy as jnp
from jax import lax
from jax.experimental import pallas as pl
from jax.experimental.pallas import tpu as pltpu

_DIM = 8
_CB = 512
_NQ = 16
_NR = 32
_LOGVAR_MIN = -30.0
_LOGVAR_MAX = 20.0
_KL_SCALE = 1.4426 * 0.5


def _body(z_ref, w_ref, p_ref, zhat_ref, idx_ref, kl_ref):
    zb = z_ref[0, 0]
    mu = zb[0:_DIM, :]
    lv = zb[_DIM:2 * _DIM, :]
    lvc = jnp.clip(lv, _LOGVAR_MIN, _LOGVAR_MAX)
    var = jnp.exp(lvc)
    inv = 1.0 / var
    acts = jnp.concatenate([0.5 - 0.5 * inv, mu * inv], axis=0)
    a_hi = acts.astype(jnp.bfloat16)
    a_lo = (acts - a_hi.astype(jnp.float32)).astype(jnp.bfloat16)
    a_big = jnp.concatenate([a_hi, a_hi, a_lo, a_lo], axis=0)
    score = lax.dot_general(
        w_ref[...], a_big, (((1,), (0,)), ((), ())),
        preferred_element_type=jnp.float32)
    blk = score.shape[1]
    iota = lax.broadcasted_iota(jnp.int32, (_CB, blk), 0)
    mx = jnp.max(score, axis=0, keepdims=True)
    idx = jnp.min(jnp.where(score == mx, iota, _CB), axis=0)
    idx_ref[0, 0, 0, :] = idx

    r = idx & (_NR - 1)
    q = idx >> 5
    iota_r = lax.broadcasted_iota(jnp.int32, (_NR, blk), 0)
    ohr = (iota_r == r[None, :]).astype(jnp.bfloat16)
    u_both = lax.dot_general(
        p_ref[...], ohr, (((1,), (0,)), ((), ())),
        preferred_element_type=jnp.float32)
    u = u_both[0:_NQ * _DIM, :] + u_both[_NQ * _DIM:2 * _NQ * _DIM, :]
    acc = jnp.zeros((_DIM, blk), jnp.float32)
    for qq in range(_NQ):
        sel = (q == qq)[None, :]
        acc = jnp.where(sel, u[qq * _DIM:(qq + 1) * _DIM, :], acc)
    zhat_ref[0, 0] = acc

    part = jnp.sum(mu * mu + var - 1.0 - lvc)

    @pl.when(jnp.logical_and(
        jnp.logical_and(pl.program_id(0) == 0, pl.program_id(1) == 0),
        pl.program_id(2) == 0))
    def _init():
        kl_ref[...] = jnp.zeros_like(kl_ref)

    kl_ref[...] += part


def _codebook_mats(prior_samples):
    ps = prior_samples.astype(jnp.float32)
    w = jnp.concatenate([ps * ps, ps], axis=1)
    w_hi = w.astype(jnp.bfloat16)
    w_lo = (w - w_hi.astype(jnp.float32)).astype(jnp.bfloat16)
    w_big = jnp.concatenate([w_hi, w_lo, w_hi, w_lo], axis=1)
    pfac = ps.reshape(_NQ, _NR, _DIM).transpose(0, 2, 1).reshape(_NQ * _DIM, _NR)
    p_hi = pfac.astype(jnp.bfloat16)
    p_lo = (pfac - p_hi.astype(jnp.float32)).astype(jnp.bfloat16)
    p_both = jnp.concatenate([p_hi, p_lo], axis=0)
    return w_big, p_both


def kernel(z, prior_samples):
    batch, chans, hh, ww = z.shape
    spatial = hh * ww
    blk = 512
    zr = z.reshape(batch, chans // 2, 2, spatial).transpose(0, 2, 1, 3)
    w_big, p_both = _codebook_mats(prior_samples)
    grid = (batch, 2, spatial // blk)
    zhat4, idx4, klsum = pl.pallas_call(
        _body,
        grid=grid,
        in_specs=[
            pl.BlockSpec((1, 1, chans // 2, blk), lambda b, g, s: (b, g, 0, s)),
            pl.BlockSpec((_CB, 64), lambda b, g, s: (0, 0)),
            pl.BlockSpec((2 * _NQ * _DIM, _NR), lambda b, g, s: (0, 0)),
        ],
        out_specs=[
            pl.BlockSpec((1, 1, _DIM, blk), lambda b, g, s: (b, g, 0, s)),
            pl.BlockSpec((1, 1, 1, blk), lambda b, g, s: (b, g, 0, s)),
            pl.BlockSpec((1, 1), lambda b, g, s: (0, 0)),
        ],
        out_shape=[
            jax.ShapeDtypeStruct((batch, 2, _DIM, spatial), jnp.float32),
            jax.ShapeDtypeStruct((batch, 2, 1, spatial), jnp.int32),
            jax.ShapeDtypeStruct((1, 1), jnp.float32),
        ],
    )(zr, w_big, p_both)
    zhat = zhat4.transpose(0, 2, 1, 3).reshape(batch, 16, hh, ww)
    indices = idx4.reshape(batch, 2, hh, ww)
    kl_loss = klsum[0, 0] * jnp.float32(_KL_SCALE / (batch * spatial * 2))
    return (zhat, kl_loss, indices)


# speedup vs baseline: 12.7068x; 1.5805x over previous
import jax
import jax.numpy as jnp
from jax import lax
from jax.experimental import pallas as pl
from jax.experimental.pallas import tpu as pltpu

_DIM = 8
_CB = 512
_NQ = 16
_NR = 32
_LOGVAR_MIN = -30.0
_LOGVAR_MAX = 20.0
_KL_SCALE = 1.4426 * 0.5


def _body(z_ref, w_ref, p_ref, zhat_ref, idx_ref, kl_ref):
    zb = z_ref[0]
    mu = zb[0:16, :]
    lv = zb[16:32, :]
    lvc = jnp.clip(lv, _LOGVAR_MIN, _LOGVAR_MAX)
    var = jnp.exp(lvc)
    inv = 1.0 / var
    acts = jnp.concatenate([0.5 - 0.5 * inv, mu * inv], axis=0)
    a_hi = acts.astype(jnp.bfloat16)
    a_lo = (acts - a_hi.astype(jnp.float32)).astype(jnp.bfloat16)
    a_big = jnp.concatenate([a_hi, a_hi, a_lo, a_lo], axis=0)
    score = lax.dot_general(
        w_ref[...], a_big, (((1,), (0,)), ((), ())),
        preferred_element_type=jnp.float32)
    blk = score.shape[1]
    iota = lax.broadcasted_iota(jnp.int32, (_CB, blk), 0)
    s0 = score[0:_CB, :]
    s1 = score[_CB:2 * _CB, :]
    m0 = jnp.max(s0, axis=0, keepdims=True)
    m1 = jnp.max(s1, axis=0, keepdims=True)
    idx0 = jnp.min(jnp.where(s0 == m0, iota, _CB), axis=0)
    idx1 = jnp.min(jnp.where(s1 == m1, iota, _CB), axis=0)
    idx_ref[0, 0, :] = idx0
    idx_ref[0, 1, :] = idx1

    iota_r = lax.broadcasted_iota(jnp.int32, (_NR, blk), 0)
    ohr0 = (iota_r == (idx0 & (_NR - 1))[None, :]).astype(jnp.bfloat16)
    ohr1 = (iota_r == (idx1 & (_NR - 1))[None, :]).astype(jnp.bfloat16)
    ohfull = jnp.concatenate([ohr0, ohr1, ohr0, ohr1], axis=0)
    u_int = lax.dot_general(
        p_ref[...], ohfull, (((1,), (0,)), ((), ())),
        preferred_element_type=jnp.float32)
    parity = lax.broadcasted_iota(jnp.int32, (16, blk), 0) & 1
    q0_b = jnp.broadcast_to((idx0 >> 5)[None, :], (16, blk))
    q1_b = jnp.broadcast_to((idx1 >> 5)[None, :], (16, blk))
    q_int = jnp.where(parity == 1, q1_b, q0_b)
    acc = jnp.zeros((16, blk), jnp.float32)
    for qq in range(_NQ):
        acc = jnp.where(q_int == qq, u_int[qq * 16:(qq + 1) * 16, :], acc)
    zhat_ref[0] = acc

    part = jnp.sum(mu * mu + var - 1.0 - lvc)

    @pl.when(jnp.logical_and(pl.program_id(0) == 0, pl.program_id(1) == 0))
    def _init():
        kl_ref[...] = jnp.zeros_like(kl_ref)

    kl_ref[...] += part


def _codebook_mats(prior_samples):
    ps = prior_samples.astype(jnp.float32)
    ps2 = ps * ps
    j = jnp.arange(16)
    dsel = j // 2
    par = j % 2
    gsel = jnp.arange(2)[:, None, None]
    wa = jnp.where(par[None, None, :] == gsel, ps2[:, dsel][None], 0.0)
    wb = jnp.where(par[None, None, :] == gsel, ps[:, dsel][None], 0.0)
    w_int = jnp.concatenate(
        [wa.reshape(2 * _CB, 16), wb.reshape(2 * _CB, 16)], axis=1)
    w_hi = w_int.astype(jnp.bfloat16)
    w_lo = (w_int - w_hi.astype(jnp.float32)).astype(jnp.bfloat16)
    w_big = jnp.concatenate([w_hi, w_lo, w_hi, w_lo], axis=1)

    arr = ps.reshape(_NQ, _NR, _DIM)
    pq = arr[:, :, dsel]
    parr = (jnp.arange(2)[None, :, None, None] == par[None, None, None, :])
    p2 = jnp.where(parr, pq[:, None, :, :], 0.0)
    p_base = p2.transpose(0, 3, 1, 2).reshape(_NQ * 16, 2 * _NR)
    p_hi = p_base.astype(jnp.bfloat16)
    p_lo = (p_base - p_hi.astype(jnp.float32)).astype(jnp.bfloat16)
    p_full = jnp.concatenate([p_hi, p_lo], axis=1)
    return w_big, p_full


def kernel(z, prior_samples):
    batch, chans, hh, ww = z.shape
    spatial = hh * ww
    blk = 512
    zr = z.reshape(batch, chans, spatial)
    w_big, p_full = _codebook_mats(prior_samples)
    grid = (batch, spatial // blk)
    zhat3, idx3, klsum = pl.pallas_call(
        _body,
        grid=grid,
        in_specs=[
            pl.BlockSpec((1, chans, blk), lambda b, s: (b, 0, s)),
            pl.BlockSpec((2 * _CB, 128), lambda b, s: (0, 0)),
            pl.BlockSpec((_NQ * 16, 128), lambda b, s: (0, 0)),
        ],
        out_specs=[
            pl.BlockSpec((1, 16, blk), lambda b, s: (b, 0, s)),
            pl.BlockSpec((1, 2, blk), lambda b, s: (b, 0, s)),
            pl.BlockSpec((1, 1), lambda b, s: (0, 0)),
        ],
        out_shape=[
            jax.ShapeDtypeStruct((batch, 16, spatial), jnp.float32),
            jax.ShapeDtypeStruct((batch, 2, spatial), jnp.int32),
            jax.ShapeDtypeStruct((1, 1), jnp.float32),
        ],
    )(zr, w_big, p_full)
    zhat = zhat3.reshape(batch, 16, hh, ww)
    indices = idx3.reshape(batch, 2, hh, ww)
    kl_loss = klsum[0, 0] * jnp.float32(_KL_SCALE / (batch * spatial * 2))
    return (zhat, kl_loss, indices)


# speedup vs baseline: 15.0975x; 1.1881x over previous
import jax
import jax.numpy as jnp
from jax import lax
from jax.experimental import pallas as pl
from jax.experimental.pallas import tpu as pltpu

_DIM = 8
_CB = 512
_NQ = 16
_NR = 32
_LOGVAR_MIN = -30.0
_LOGVAR_MAX = 20.0
_KL_SCALE = 1.4426 * 0.5


def _body(z_ref, w_ref, p_ref, zhat_ref, idx_ref, kl_ref):
    zb = z_ref[0]
    mu = zb[0:16, :]
    lv = zb[16:32, :]
    lvc = jnp.clip(lv, _LOGVAR_MIN, _LOGVAR_MAX)
    var = jnp.exp(lvc)
    inv = 1.0 / var
    acts = jnp.concatenate([0.5 - 0.5 * inv, mu * inv], axis=0)
    a_hi = acts.astype(jnp.bfloat16)
    a_lo = (acts - a_hi.astype(jnp.float32)).astype(jnp.bfloat16)
    a_big = jnp.concatenate([a_hi, a_hi, a_lo, a_lo], axis=0)
    score = lax.dot_general(
        w_ref[...], a_big, (((1,), (0,)), ((), ())),
        preferred_element_type=jnp.float32)
    blk = score.shape[1]
    iota = lax.broadcasted_iota(jnp.int32, (_CB, blk), 0)
    s0 = score[0:_CB, :]
    s1 = score[_CB:2 * _CB, :]
    m0 = jnp.max(s0, axis=0, keepdims=True)
    m1 = jnp.max(s1, axis=0, keepdims=True)
    idx0 = jnp.min(jnp.where(s0 == m0, iota, _CB), axis=0)
    idx1 = jnp.min(jnp.where(s1 == m1, iota, _CB), axis=0)
    idx_ref[0, 0, :] = idx0
    idx_ref[0, 1, :] = idx1

    iota_r = lax.broadcasted_iota(jnp.int32, (_NR, blk), 0)
    ohr0 = (iota_r == (idx0 & (_NR - 1))[None, :]).astype(jnp.bfloat16)
    ohr1 = (iota_r == (idx1 & (_NR - 1))[None, :]).astype(jnp.bfloat16)
    ohfull = jnp.concatenate([ohr0, ohr1, ohr0, ohr1], axis=0)
    u_int = lax.dot_general(
        p_ref[...], ohfull, (((1,), (0,)), ((), ())),
        preferred_element_type=jnp.float32)
    parity = lax.broadcasted_iota(jnp.int32, (16, blk), 0) & 1
    q0_b = jnp.broadcast_to((idx0 >> 5)[None, :], (16, blk))
    q1_b = jnp.broadcast_to((idx1 >> 5)[None, :], (16, blk))
    q_int = jnp.where(parity == 1, q1_b, q0_b)
    acc = jnp.zeros((16, blk), jnp.float32)
    for qq in range(_NQ):
        acc = jnp.where(q_int == qq, u_int[qq * 16:(qq + 1) * 16, :], acc)
    zhat_ref[0] = acc

    part = jnp.sum(mu * mu + var - 1.0 - lvc)

    @pl.when(jnp.logical_and(pl.program_id(0) == 0, pl.program_id(1) == 0))
    def _init():
        kl_ref[...] = jnp.zeros_like(kl_ref)

    kl_ref[...] += part


def _codebook_mats(prior_samples):
    ps = prior_samples.astype(jnp.float32)
    ps2 = ps * ps
    j = jnp.arange(16)
    dsel = j // 2
    par = j % 2
    gsel = jnp.arange(2)[:, None, None]
    wa = jnp.where(par[None, None, :] == gsel, ps2[:, dsel][None], 0.0)
    wb = jnp.where(par[None, None, :] == gsel, ps[:, dsel][None], 0.0)
    w_int = jnp.concatenate(
        [wa.reshape(2 * _CB, 16), wb.reshape(2 * _CB, 16)], axis=1)
    w_hi = w_int.astype(jnp.bfloat16)
    w_lo = (w_int - w_hi.astype(jnp.float32)).astype(jnp.bfloat16)
    w_big = jnp.concatenate([w_hi, w_lo, w_hi, w_lo], axis=1)

    arr = ps.reshape(_NQ, _NR, _DIM)
    pq = arr[:, :, dsel]
    parr = (jnp.arange(2)[None, :, None, None] == par[None, None, None, :])
    p2 = jnp.where(parr, pq[:, None, :, :], 0.0)
    p_base = p2.transpose(0, 3, 1, 2).reshape(_NQ * 16, 2 * _NR)
    p_hi = p_base.astype(jnp.bfloat16)
    p_lo = (p_base - p_hi.astype(jnp.float32)).astype(jnp.bfloat16)
    p_full = jnp.concatenate([p_hi, p_lo], axis=1)
    return w_big, p_full


def kernel(z, prior_samples):
    batch, chans, hh, ww = z.shape
    spatial = hh * ww
    blk = 1024
    zr = z.reshape(batch, chans, spatial)
    w_big, p_full = _codebook_mats(prior_samples)
    grid = (batch, spatial // blk)
    zhat3, idx3, klsum = pl.pallas_call(
        _body,
        grid=grid,
        in_specs=[
            pl.BlockSpec((1, chans, blk), lambda b, s: (b, 0, s)),
            pl.BlockSpec((2 * _CB, 128), lambda b, s: (0, 0)),
            pl.BlockSpec((_NQ * 16, 128), lambda b, s: (0, 0)),
        ],
        out_specs=[
            pl.BlockSpec((1, 16, blk), lambda b, s: (b, 0, s)),
            pl.BlockSpec((1, 2, blk), lambda b, s: (b, 0, s)),
            pl.BlockSpec((1, 1), lambda b, s: (0, 0)),
        ],
        out_shape=[
            jax.ShapeDtypeStruct((batch, 16, spatial), jnp.float32),
            jax.ShapeDtypeStruct((batch, 2, spatial), jnp.int32),
            jax.ShapeDtypeStruct((1, 1), jnp.float32),
        ],
    )(zr, w_big, p_full)
    zhat = zhat3.reshape(batch, 16, hh, ww)
    indices = idx3.reshape(batch, 2, hh, ww)
    kl_loss = klsum[0, 0] * jnp.float32(_KL_SCALE / (batch * spatial * 2))
    return (zhat, kl_loss, indices)
